# Initial kernel scaffold; baseline (speedup 1.0000x reference)
import functools

import jax
import jax.numpy as jnp
from jax import lax
from jax.experimental import pallas as pl
from jax.experimental.pallas import tpu as pltpu
from jax.experimental.pallas import tpu_sc as plsc

F = 26
D = 16
BUCKET = 1020
FIELD_DIM = 40000
B = 16384
NW = 32
BPW = B // NW
NBLK = BPW // 16
LIN_CHUNK = 128
NCHUNK = F * BPW // LIN_CHUNK


def _thresh_body(q_ref, r_ref, s_ref, oq_ref, or_ref):
    tq = jax.nn.sigmoid(s_ref[0, 0])
    tr = jax.nn.sigmoid(s_ref[0, 1])
    q = q_ref[...]
    r = r_ref[...]
    oq_ref[...] = jnp.sign(q) * jnp.maximum(jnp.abs(q) - tq, 0.0)
    or_ref[...] = jnp.sign(r) * jnp.maximum(jnp.abs(r) - tr, 0.0)


def _threshold_tables(Q_v, R_v, Q_s, R_s):
    scal = jnp.stack([Q_s, R_s]).reshape(1, 2).astype(jnp.float32)
    return pl.pallas_call(
        _thresh_body,
        out_shape=(
            jax.ShapeDtypeStruct(Q_v.shape, jnp.float32),
            jax.ShapeDtypeStruct(R_v.shape, jnp.float32),
        ),
        in_specs=[
            pl.BlockSpec(memory_space=pltpu.VMEM),
            pl.BlockSpec(memory_space=pltpu.VMEM),
            pl.BlockSpec(memory_space=pltpu.SMEM),
        ],
        out_specs=(
            pl.BlockSpec(memory_space=pltpu.VMEM),
            pl.BlockSpec(memory_space=pltpu.VMEM),
        ),
    )(Q_v, R_v, scal)


def _sc_body(xT_hbm, qt_hbm, rt_hbm, bias_hbm, lin_hbm, out_hbm,
             x_v, qtab, rtab, bias_v, qbuf, rbuf, gidx, linbuf, out_v, sem):
    cid = lax.axis_index("c")
    sid = lax.axis_index("s")
    wid = sid * 2 + cid
    base = wid * BPW

    pltpu.sync_copy(xT_hbm.at[:, pl.ds(base, BPW)], x_v)
    pltpu.sync_copy(qt_hbm, qtab)
    pltpu.sync_copy(rt_hbm, rtab)
    pltpu.sync_copy(bias_hbm, bias_v)

    def prep(i, _):
        f = i // NBLK
        s0 = (i % NBLK) * 16
        g = x_v[f, pl.ds(s0, 16)] + f * FIELD_DIM
        q = lax.div(g, BUCKET)
        r = g - q * BUCKET
        qbuf[f, pl.ds(s0, 16)] = q
        rbuf[f, pl.ds(s0, 16)] = r
        p = i * 16
        gidx[p // LIN_CHUNK, pl.ds(p % LIN_CHUNK, 16)] = g
        return 0

    lax.fori_loop(0, F * NBLK, prep, 0)

    def fire(j, _):
        pltpu.make_async_copy(lin_hbm.at[gidx.at[j]], linbuf.at[j], sem).start()
        return 0

    lax.fori_loop(0, NCHUNK, fire, 0)

    def fm(blk, _):
        s0 = blk * 16
        acc_s = [jnp.zeros((16,), jnp.float32) for _ in range(D)]
        acc_q = [jnp.zeros((16,), jnp.float32) for _ in range(D)]
        for f in range(F):
            qi = qbuf[f, pl.ds(s0, 16)]
            ri = rbuf[f, pl.ds(s0, 16)]
            for d in range(D):
                col = jnp.full((16,), d, jnp.int32)
                qv = plsc.load_gather(qtab, [qi, col])
                rv = plsc.load_gather(rtab, [ri, col])
                e = qv * rv
                acc_s[d] = acc_s[d] + e
                acc_q[d] = acc_q[d] + e * e
        tot = acc_s[0] * acc_s[0] - acc_q[0]
        for d in range(1, D):
            tot = tot + (acc_s[d] * acc_s[d] - acc_q[d])
        out_v[pl.ds(s0, 16)] = 0.5 * tot
        return 0

    lax.fori_loop(0, NBLK, fm, 0)

    def drain(j, _):
        pltpu.make_async_copy(lin_hbm.at[gidx.at[j]], linbuf.at[j], sem).wait()
        return 0

    lax.fori_loop(0, NCHUNK, drain, 0)

    def lin(blk, _):
        s0 = blk * 16
        acc = bias_v[:]
        for f in range(F):
            row = (f * BPW) // LIN_CHUNK
            acc = acc + linbuf[row + blk // 8, pl.ds((blk % 8) * 16, 16)]
        out_v[pl.ds(s0, 16)] = out_v[pl.ds(s0, 16)] + acc
        return 0

    lax.fori_loop(0, NBLK, lin, 0)

    pltpu.sync_copy(out_v, out_hbm.at[pl.ds(base, BPW)])


@functools.partial(
    pl.kernel,
    out_type=jax.ShapeDtypeStruct((B,), jnp.float32),
    mesh=plsc.VectorSubcoreMesh(core_axis_name="c", subcore_axis_name="s"),
    scratch_types=[
        pltpu.VMEM((F, BPW), jnp.int32),
        pltpu.VMEM((BUCKET, D), jnp.float32),
        pltpu.VMEM((BUCKET, D), jnp.float32),
        pltpu.VMEM((16,), jnp.float32),
        pltpu.VMEM((F, BPW), jnp.int32),
        pltpu.VMEM((F, BPW), jnp.int32),
        pltpu.VMEM((NCHUNK, LIN_CHUNK), jnp.int32),
        pltpu.VMEM((NCHUNK, LIN_CHUNK), jnp.float32),
        pltpu.VMEM((BPW,), jnp.float32),
        pltpu.SemaphoreType.DMA,
    ],
)
def _sc_kernel(*refs):
    _sc_body(*refs)


def kernel(x, Q_v, R_v, Q_s, R_s, lin_w, lin_b, offsets):
    del offsets
    qt, rt = _threshold_tables(Q_v, R_v, Q_s, R_s)
    xT = x.T
    bias_v = jnp.full((16,), lin_b[0], jnp.float32)
    lin_flat = lin_w.reshape(-1)
    return _sc_kernel(xT, qt, rt, bias_v, lin_flat)


# speedup vs baseline: 16.0682x; 16.0682x over previous
import functools

import jax
import jax.numpy as jnp
from jax import lax
from jax.experimental import pallas as pl
from jax.experimental.pallas import tpu as pltpu
from jax.experimental.pallas import tpu_sc as plsc

F = 26
D = 16
BUCKET = 1020
FIELD_DIM = 40000
B = 16384
NW = 32
BPW = B // NW
NBLK = BPW // 16
LIN_CHUNK = 128
NCHUNK = F * BPW // LIN_CHUNK


def _thresh_body(q_ref, r_ref, s_ref, oq_ref, or_ref):
    tq = jax.nn.sigmoid(s_ref[0, 0])
    tr = jax.nn.sigmoid(s_ref[0, 1])
    q = q_ref[...]
    r = r_ref[...]
    oq_ref[...] = jnp.sign(q) * jnp.maximum(jnp.abs(q) - tq, 0.0)
    or_ref[...] = jnp.sign(r) * jnp.maximum(jnp.abs(r) - tr, 0.0)


def _threshold_tables(Q_v, R_v, Q_s, R_s):
    scal = jnp.stack([Q_s, R_s]).reshape(1, 2).astype(jnp.float32)
    return pl.pallas_call(
        _thresh_body,
        out_shape=(
            jax.ShapeDtypeStruct(Q_v.shape, jnp.float32),
            jax.ShapeDtypeStruct(R_v.shape, jnp.float32),
        ),
        in_specs=[
            pl.BlockSpec(memory_space=pltpu.VMEM),
            pl.BlockSpec(memory_space=pltpu.VMEM),
            pl.BlockSpec(memory_space=pltpu.SMEM),
        ],
        out_specs=(
            pl.BlockSpec(memory_space=pltpu.VMEM),
            pl.BlockSpec(memory_space=pltpu.VMEM),
        ),
    )(Q_v, R_v, scal)


def _sc_body(xT_hbm, qt_hbm, rt_hbm, bias_hbm, lin_hbm, out_hbm,
             x_v, qtab, rtab, bias_v, qbuf, rbuf, gidx, linbuf, out_v, sem):
    cid = lax.axis_index("c")
    sid = lax.axis_index("s")
    wid = sid * 2 + cid
    base = wid * BPW

    pltpu.sync_copy(xT_hbm.at[:, pl.ds(base, BPW)], x_v)
    pltpu.sync_copy(qt_hbm, qtab)
    pltpu.sync_copy(rt_hbm, rtab)
    pltpu.sync_copy(bias_hbm, bias_v)

    def prep(i, _):
        f = i // NBLK
        s0 = (i % NBLK) * 16
        g = x_v[f, pl.ds(s0, 16)] + f * FIELD_DIM
        q = lax.div(g, BUCKET)
        r = g - q * BUCKET
        qbuf[f, pl.ds(s0, 16)] = q
        rbuf[f, pl.ds(s0, 16)] = r
        p = i * 16
        gidx[p // LIN_CHUNK, pl.ds(p % LIN_CHUNK, 16)] = g
        return 0

    lax.fori_loop(0, F * NBLK, prep, 0)

    def fire(j, _):
        pltpu.make_async_copy(lin_hbm.at[gidx.at[j]], linbuf.at[j], sem).start()
        return 0

    lax.fori_loop(0, NCHUNK, fire, 0)

    def fm(blk, _):
        s0 = blk * 16
        acc_s = [jnp.zeros((16,), jnp.float32) for _ in range(D)]
        acc_q = [jnp.zeros((16,), jnp.float32) for _ in range(D)]
        for f in range(F):
            qi = qbuf[f, pl.ds(s0, 16)] * D
            ri = rbuf[f, pl.ds(s0, 16)] * D
            for d in range(D):
                qv = plsc.load_gather(qtab, [qi + d])
                rv = plsc.load_gather(rtab, [ri + d])
                e = qv * rv
                acc_s[d] = acc_s[d] + e
                acc_q[d] = acc_q[d] + e * e
        tot = acc_s[0] * acc_s[0] - acc_q[0]
        for d in range(1, D):
            tot = tot + (acc_s[d] * acc_s[d] - acc_q[d])
        out_v[pl.ds(s0, 16)] = 0.5 * tot
        return 0

    lax.fori_loop(0, NBLK, fm, 0)

    def drain(j, _):
        pltpu.make_async_copy(lin_hbm.at[gidx.at[j]], linbuf.at[j], sem).wait()
        return 0

    lax.fori_loop(0, NCHUNK, drain, 0)

    def lin(blk, _):
        s0 = blk * 16
        acc = bias_v[:]
        for f in range(F):
            row = (f * BPW) // LIN_CHUNK
            acc = acc + linbuf[row + blk // 8, pl.ds((blk % 8) * 16, 16)]
        out_v[pl.ds(s0, 16)] = out_v[pl.ds(s0, 16)] + acc
        return 0

    lax.fori_loop(0, NBLK, lin, 0)

    pltpu.sync_copy(out_v, out_hbm.at[pl.ds(base, BPW)])


@functools.partial(
    pl.kernel,
    out_type=jax.ShapeDtypeStruct((B,), jnp.float32),
    mesh=plsc.VectorSubcoreMesh(core_axis_name="c", subcore_axis_name="s"),
    compiler_params=pltpu.CompilerParams(needs_layout_passes=False),
    scratch_types=[
        pltpu.VMEM((F, BPW), jnp.int32),
        pltpu.VMEM((BUCKET * D,), jnp.float32),
        pltpu.VMEM((BUCKET * D,), jnp.float32),
        pltpu.VMEM((16,), jnp.float32),
        pltpu.VMEM((F, BPW), jnp.int32),
        pltpu.VMEM((F, BPW), jnp.int32),
        pltpu.VMEM((NCHUNK, LIN_CHUNK), jnp.int32),
        pltpu.VMEM((NCHUNK, LIN_CHUNK), jnp.float32),
        pltpu.VMEM((BPW,), jnp.float32),
        pltpu.SemaphoreType.DMA,
    ],
)
def _sc_kernel(*refs):
    _sc_body(*refs)


def kernel(x, Q_v, R_v, Q_s, R_s, lin_w, lin_b, offsets):
    del offsets
    qt, rt = _threshold_tables(Q_v, R_v, Q_s, R_s)
    xT = x.T
    bias_v = jnp.full((16,), lin_b[0], jnp.float32)
    lin_flat = lin_w.reshape(-1)
    return _sc_kernel(xT, qt.reshape(-1), rt.reshape(-1), bias_v, lin_flat)
